# baseline (speedup 1.0000x reference)
import jax
import jax.numpy as jnp
from jax import lax
from jax.experimental import pallas as pl
from jax.experimental.pallas import tpu as pltpu
from jax.experimental.pallas import tpu_sc as plsc

N_NODES = 10000
N_EDGES = 320000
IN_CH = 128
OUT_CH = 128
HCH = OUT_CH // 2

NC = 2
NS = 16
CHUNK = 128

CHUNKS_PER_TILE = 160
EDGES_PER_TILE = CHUNKS_PER_TILE * CHUNK
E_PAD = EDGES_PER_TILE * NS
NBUF = 4

N_DUMMY = 8
N_ROWS = N_NODES + N_DUMMY

RPT = 624
REM0 = RPT * NS


def _mm_body(x_ref, nrm_ref, w_ref, o_ref):
    o_ref[0] = jnp.dot(
        x_ref[...] * nrm_ref[...], w_ref[0],
        preferred_element_type=jnp.float32,
    )


def _finish_body(acc_ref, nrm_ref, b_ref, o_ref):
    o_ref[...] = (
        jnp.concatenate([acc_ref[0], acc_ref[1]], axis=1) * nrm_ref[...]
        + b_ref[...]
    )


def _mm_call(x, norm, w2):
    R = 1000
    return pl.pallas_call(
        _mm_body,
        grid=(NC, N_NODES // R),
        in_specs=[
            pl.BlockSpec((R, IN_CH), lambda j, i: (i, 0)),
            pl.BlockSpec((R, 1), lambda j, i: (i, 0)),
            pl.BlockSpec((1, IN_CH, HCH), lambda j, i: (j, 0, 0)),
        ],
        out_specs=pl.BlockSpec((1, R, HCH), lambda j, i: (j, i, 0)),
        out_shape=jax.ShapeDtypeStruct((NC, N_ROWS, HCH), jnp.float32),
    )(x, norm, w2)


def _scatter_body(h2_hbm, idxp_hbm, zeros_hbm, out_hbm,
                  h_sp, acc, idxpb, sbufs, dbufs, msgs, lsems, gsems, ssems):
    c = lax.axis_index("c")
    s = lax.axis_index("s")

    r0 = s * RPT
    pltpu.sync_copy(zeros_hbm.at[pl.ds(r0, RPT)], acc.at[pl.ds(r0, RPT)])
    pltpu.sync_copy(h2_hbm.at[c, pl.ds(r0, RPT)], h_sp.at[pl.ds(r0, RPT)])

    @pl.when(s == 0)
    def _():
        pltpu.sync_copy(zeros_hbm.at[pl.ds(REM0, N_ROWS - REM0)],
                        acc.at[pl.ds(REM0, N_ROWS - REM0)])
        pltpu.sync_copy(h2_hbm.at[c, pl.ds(REM0, N_ROWS - REM0)],
                        h_sp.at[pl.ds(REM0, N_ROWS - REM0)])

    plsc.subcore_barrier()

    def load(ci, b):
        return pltpu.make_async_copy(idxp_hbm.at[s, ci], idxpb.at[b],
                                     lsems.at[b])

    def unpack_src(b):
        row = idxpb.at[b]
        sb = sbufs.at[b]
        for j in range(CHUNK // 16):
            v = row[pl.ds(j * 16, 16)]
            sb[pl.ds(j * 16, 16)] = jnp.bitwise_and(v, 0xFFFF)

    def unpack_dst(b):
        row = idxpb.at[b]
        db = dbufs.at[b]
        for j in range(CHUNK // 16):
            v = row[pl.ds(j * 16, 16)]
            db[pl.ds(j * 16, 16)] = jnp.right_shift(v, 16)

    def gather(b):
        return pltpu.make_async_copy(h_sp.at[sbufs.at[b]], msgs.at[b],
                                     gsems.at[b])

    def scatter(b):
        return pltpu.make_async_copy(msgs.at[b], acc.at[dbufs.at[b]],
                                     ssems.at[b])

    for b in range(NBUF):
        load(b, b).start()

    def body(k, carry):
        c0 = k * NBUF
        for b in range(NBUF):
            @pl.when(k > 0)
            def _():
                scatter(b).wait()

            load(c0 + b, b).wait()
            unpack_src(b)
            unpack_dst(b)

            @pl.when(c0 + b + NBUF < CHUNKS_PER_TILE)
            def _():
                load(c0 + b + NBUF, b).start()

            gather(b).start()
        for b in range(NBUF):
            gather(b).wait()
            scatter(b).start(add=True)
        return carry

    lax.fori_loop(0, CHUNKS_PER_TILE // NBUF, body, 0)
    for b in range(NBUF):
        scatter(b).wait()
    plsc.subcore_barrier()

    pltpu.sync_copy(acc.at[pl.ds(r0, RPT)], out_hbm.at[c, pl.ds(r0, RPT)])

    @pl.when(s == 0)
    def _():
        pltpu.sync_copy(acc.at[pl.ds(REM0, N_NODES - REM0)],
                        out_hbm.at[c, pl.ds(REM0, N_NODES - REM0)])


@jax.jit
def kernel(x, edge_index, norm, weight, bias):
    x = x.astype(jnp.float32)
    norm = norm.astype(jnp.float32)

    src = edge_index[0].astype(jnp.int32)
    dst = edge_index[1].astype(jnp.int32)
    npad = E_PAD - N_EDGES
    src = jnp.concatenate([src, jnp.zeros((npad,), jnp.int32)])
    dst = jnp.concatenate(
        [dst, N_NODES + (jnp.arange(npad, dtype=jnp.int32) % N_DUMMY)])
    idxp = (src + dst * 65536).reshape(NS, CHUNKS_PER_TILE, CHUNK)

    w2 = jnp.moveaxis(weight.reshape(IN_CH, NC, HCH), 1, 0)
    h2 = _mm_call(x, norm, w2)

    zeros = jnp.zeros((N_ROWS, HCH), jnp.float32)
    mesh = plsc.VectorSubcoreMesh(core_axis_name="c", subcore_axis_name="s")
    acc2 = pl.kernel(
        _scatter_body,
        out_type=jax.ShapeDtypeStruct((NC, N_NODES, HCH), jnp.float32),
        mesh=mesh,
        compiler_params=pltpu.CompilerParams(use_tc_tiling_on_sc=False),
        scratch_types=[
            pltpu.VMEM_SHARED((N_ROWS, HCH), jnp.float32),
            pltpu.VMEM_SHARED((N_ROWS, HCH), jnp.float32),
            pltpu.VMEM((NBUF, CHUNK), jnp.int32),
            pltpu.VMEM((NBUF, CHUNK), jnp.int32),
            pltpu.VMEM((NBUF, CHUNK), jnp.int32),
            pltpu.VMEM((NBUF, CHUNK, HCH), jnp.float32),
            pltpu.SemaphoreType.DMA((NBUF,)),
            pltpu.SemaphoreType.DMA((NBUF,)),
            pltpu.SemaphoreType.DMA((NBUF,)),
        ],
    )(h2, idxp, zeros)

    R = 1000
    out = pl.pallas_call(
        _finish_body,
        grid=(N_NODES // R,),
        in_specs=[
            pl.BlockSpec((NC, R, HCH), lambda i: (0, i, 0)),
            pl.BlockSpec((R, 1), lambda i: (i, 0)),
            pl.BlockSpec((1, OUT_CH), lambda i: (0, 0)),
        ],
        out_specs=pl.BlockSpec((R, OUT_CH), lambda i: (i, 0)),
        out_shape=jax.ShapeDtypeStruct((N_NODES, OUT_CH), jnp.float32),
    )(acc2, norm, bias.reshape(1, OUT_CH))
    return out


# speedup vs baseline: 1.1343x; 1.1343x over previous
import jax
import jax.numpy as jnp
from jax import lax
from jax.experimental import pallas as pl
from jax.experimental.pallas import tpu as pltpu
from jax.experimental.pallas import tpu_sc as plsc

N_NODES = 10000
N_EDGES = 320000
IN_CH = 128
OUT_CH = 128
HCH = OUT_CH // 2

NC = 2
NS = 16
CHUNK = 128

CHUNKS_PER_TILE = 160
EDGES_PER_TILE = CHUNKS_PER_TILE * CHUNK
E_PAD = EDGES_PER_TILE * NS
GROUPS = CHUNKS_PER_TILE // 2

N_DUMMY = 8
N_ROWS = N_NODES + N_DUMMY

RPT = 624
REM0 = RPT * NS


def _mm_body(x_ref, nrm_ref, w_ref, s_ref, d_ref, o_ref, ip_ref, z_ref):
    o_ref[0] = jnp.dot(
        x_ref[...] * nrm_ref[...], w_ref[0],
        preferred_element_type=jnp.float32,
    )
    ip_ref[...] = s_ref[...] + jnp.left_shift(d_ref[...], 16)
    z_ref[...] = jnp.zeros_like(z_ref)


def _finish_body(acc_ref, nrm_ref, b_ref, o_ref):
    o_ref[...] = (
        jnp.concatenate([acc_ref[0], acc_ref[1]], axis=1) * nrm_ref[...]
        + b_ref[...]
    )


def _mm_call(x, norm, w2, srcr, dstr):
    R = 1000
    nsteps = N_NODES // R
    cpg = CHUNKS_PER_TILE // (NC * nsteps)
    return pl.pallas_call(
        _mm_body,
        grid=(NC, nsteps),
        in_specs=[
            pl.BlockSpec((R, IN_CH), lambda j, i: (i, 0)),
            pl.BlockSpec((R, 1), lambda j, i: (i, 0)),
            pl.BlockSpec((1, IN_CH, HCH), lambda j, i: (j, 0, 0)),
            pl.BlockSpec((NS, cpg, CHUNK), lambda j, i: (0, j * 10 + i, 0)),
            pl.BlockSpec((NS, cpg, CHUNK), lambda j, i: (0, j * 10 + i, 0)),
        ],
        out_specs=[
            pl.BlockSpec((1, R, HCH), lambda j, i: (j, i, 0)),
            pl.BlockSpec((NS, cpg, CHUNK), lambda j, i: (0, j * 10 + i, 0)),
            pl.BlockSpec((R, HCH), lambda j, i: (i, 0)),
        ],
        out_shape=[
            jax.ShapeDtypeStruct((NC, N_ROWS, HCH), jnp.float32),
            jax.ShapeDtypeStruct((NS, CHUNKS_PER_TILE, CHUNK), jnp.int32),
            jax.ShapeDtypeStruct((N_ROWS, HCH), jnp.float32),
        ],
    )(x, norm, w2, srcr, dstr)


def _scatter_body(h2_hbm, idxp_hbm, zeros_hbm, out_hbm,
                  h_sp, acc, idxpb, sbufs, dbufs, msgs, lsems, gsems, ssems):
    c = lax.axis_index("c")
    s = lax.axis_index("s")

    r0 = s * RPT
    pltpu.sync_copy(zeros_hbm.at[pl.ds(r0, RPT)], acc.at[pl.ds(r0, RPT)])
    pltpu.sync_copy(h2_hbm.at[c, pl.ds(r0, RPT)], h_sp.at[pl.ds(r0, RPT)])

    @pl.when(s == 0)
    def _():
        pltpu.sync_copy(zeros_hbm.at[pl.ds(REM0, N_ROWS - REM0)],
                        acc.at[pl.ds(REM0, N_ROWS - REM0)])
        pltpu.sync_copy(h2_hbm.at[c, pl.ds(REM0, N_ROWS - REM0)],
                        h_sp.at[pl.ds(REM0, N_ROWS - REM0)])

    plsc.subcore_barrier()

    def load(ci, b):
        return pltpu.make_async_copy(idxp_hbm.at[s, ci], idxpb.at[b],
                                     lsems.at[b])

    def unpack_src(b):
        row = idxpb.at[b]
        sb = sbufs.at[b]
        for j in range(CHUNK // 16):
            v = row[pl.ds(j * 16, 16)]
            sb[pl.ds(j * 16, 16)] = jnp.bitwise_and(v, 0xFFFF)

    def unpack_dst(b):
        row = idxpb.at[b]
        db = dbufs.at[b]
        for j in range(CHUNK // 16):
            v = row[pl.ds(j * 16, 16)]
            db[pl.ds(j * 16, 16)] = jnp.right_shift(v, 16)

    def gather(b):
        return pltpu.make_async_copy(h_sp.at[sbufs.at[b]], msgs.at[b],
                                     gsems.at[b])

    def scatter(b):
        return pltpu.make_async_copy(msgs.at[b], acc.at[dbufs.at[b]],
                                     ssems.at[b])

    load(0, 0).start()
    load(1, 1).start()
    load(0, 0).wait()
    unpack_src(0)
    unpack_dst(0)
    load(2, 0).start()
    load(1, 1).wait()
    unpack_src(1)
    gather(0).start()

    def body(k, carry):
        c0 = 2 * k
        c1 = c0 + 1
        more = k < GROUPS - 1

        gather(0).wait()
        scatter(0).start(add=True)

        @pl.when(more)
        def _():
            load(c0 + 2, 0).wait()
            unpack_src(0)

        @pl.when(k > 0)
        def _():
            scatter(1).wait()

        unpack_dst(1)

        @pl.when(more)
        def _():
            load(c1 + 2, 1).start()

        gather(1).start()
        gather(1).wait()
        scatter(1).start(add=True)

        @pl.when(more)
        def _():
            load(c1 + 2, 1).wait()
            unpack_src(1)

        scatter(0).wait()

        @pl.when(more)
        def _():
            unpack_dst(0)
            gather(0).start()

        @pl.when(k < GROUPS - 2)
        def _():
            load(c0 + 4, 0).start()

        return carry

    lax.fori_loop(0, GROUPS, body, 0)
    scatter(1).wait()
    plsc.subcore_barrier()

    pltpu.sync_copy(acc.at[pl.ds(r0, RPT)], out_hbm.at[c, pl.ds(r0, RPT)])

    @pl.when(s == 0)
    def _():
        pltpu.sync_copy(acc.at[pl.ds(REM0, N_NODES - REM0)],
                        out_hbm.at[c, pl.ds(REM0, N_NODES - REM0)])


@jax.jit
def kernel(x, edge_index, norm, weight, bias):
    x = x.astype(jnp.float32)
    norm = norm.astype(jnp.float32)

    src = edge_index[0].astype(jnp.int32)
    dst = edge_index[1].astype(jnp.int32)
    npad = E_PAD - N_EDGES
    src = jnp.concatenate([src, jnp.zeros((npad,), jnp.int32)])
    dst = jnp.concatenate(
        [dst, N_NODES + (jnp.arange(npad, dtype=jnp.int32) % N_DUMMY)])
    srcr = src.reshape(NS, CHUNKS_PER_TILE, CHUNK)
    dstr = dst.reshape(NS, CHUNKS_PER_TILE, CHUNK)

    w2 = jnp.moveaxis(weight.reshape(IN_CH, NC, HCH), 1, 0)
    h2, idxp, zeros = _mm_call(x, norm, w2, srcr, dstr)

    mesh = plsc.VectorSubcoreMesh(core_axis_name="c", subcore_axis_name="s")
    acc2 = pl.kernel(
        _scatter_body,
        out_type=jax.ShapeDtypeStruct((NC, N_NODES, HCH), jnp.float32),
        mesh=mesh,
        compiler_params=pltpu.CompilerParams(use_tc_tiling_on_sc=False),
        scratch_types=[
            pltpu.VMEM_SHARED((N_ROWS, HCH), jnp.float32),
            pltpu.VMEM_SHARED((N_ROWS, HCH), jnp.float32),
            pltpu.VMEM((2, CHUNK), jnp.int32),
            pltpu.VMEM((2, CHUNK), jnp.int32),
            pltpu.VMEM((2, CHUNK), jnp.int32),
            pltpu.VMEM((2, CHUNK, HCH), jnp.float32),
            pltpu.SemaphoreType.DMA((2,)),
            pltpu.SemaphoreType.DMA((2,)),
            pltpu.SemaphoreType.DMA((2,)),
        ],
    )(h2, idxp, zeros)

    R = 1000
    out = pl.pallas_call(
        _finish_body,
        grid=(N_NODES // R,),
        in_specs=[
            pl.BlockSpec((NC, R, HCH), lambda i: (0, i, 0)),
            pl.BlockSpec((R, 1), lambda i: (i, 0)),
            pl.BlockSpec((1, OUT_CH), lambda i: (0, 0)),
        ],
        out_specs=pl.BlockSpec((R, OUT_CH), lambda i: (i, 0)),
        out_shape=jax.ShapeDtypeStruct((N_NODES, OUT_CH), jnp.float32),
    )(acc2, norm, bias.reshape(1, OUT_CH))
    return out


# speedup vs baseline: 1.2245x; 1.0795x over previous
import jax
import jax.numpy as jnp
from jax import lax
from jax.experimental import pallas as pl
from jax.experimental.pallas import tpu as pltpu
from jax.experimental.pallas import tpu_sc as plsc

N_NODES = 10000
N_EDGES = 320000
IN_CH = 128
OUT_CH = 128
HCH = OUT_CH // 2

NC = 2
NS = 16
CHUNK = 128

CHUNKS_PER_TILE = 158
EDGES_PER_TILE = CHUNKS_PER_TILE * CHUNK
E_PAD = EDGES_PER_TILE * NS
GROUPS = CHUNKS_PER_TILE // 2

N_DUMMY = 8
N_ROWS = N_NODES + N_DUMMY

RPT = 624
REM0 = RPT * NS


def _mm_body(x_ref, nrm_ref, w_ref, o_ref):
    o_ref[0] = jnp.dot(
        x_ref[...] * nrm_ref[...], w_ref[0],
        preferred_element_type=jnp.float32,
    )


def _finish_body(acc_ref, nrm_ref, b_ref, o_ref):
    o_ref[...] = (
        jnp.concatenate([acc_ref[0], acc_ref[1]], axis=1) * nrm_ref[...]
        + b_ref[...]
    )


def _mm_call(x, norm, w2):
    R = 2000
    return pl.pallas_call(
        _mm_body,
        grid=(NC, N_NODES // R),
        in_specs=[
            pl.BlockSpec((R, IN_CH), lambda j, i: (i, 0)),
            pl.BlockSpec((R, 1), lambda j, i: (i, 0)),
            pl.BlockSpec((1, IN_CH, HCH), lambda j, i: (j, 0, 0)),
        ],
        out_specs=pl.BlockSpec((1, R, HCH), lambda j, i: (j, i, 0)),
        out_shape=jax.ShapeDtypeStruct((NC, N_ROWS, HCH), jnp.float32),
    )(x, norm, w2)


def _scatter_body(h2_hbm, idxp_hbm, zeros_hbm, out_hbm,
                  h_sp, acc, idxpb, sbufs, dbufs, msgs, lsems, gsems, ssems):
    c = lax.axis_index("c")
    s = lax.axis_index("s")

    r0 = s * RPT
    pltpu.sync_copy(zeros_hbm.at[pl.ds(r0, RPT)], acc.at[pl.ds(r0, RPT)])
    pltpu.sync_copy(h2_hbm.at[c, pl.ds(r0, RPT)], h_sp.at[pl.ds(r0, RPT)])

    @pl.when(s == 0)
    def _():
        pltpu.sync_copy(zeros_hbm.at[pl.ds(REM0, N_ROWS - REM0)],
                        acc.at[pl.ds(REM0, N_ROWS - REM0)])
        pltpu.sync_copy(h2_hbm.at[c, pl.ds(REM0, N_ROWS - REM0)],
                        h_sp.at[pl.ds(REM0, N_ROWS - REM0)])

    plsc.subcore_barrier()

    def load(ci, b):
        return pltpu.make_async_copy(idxp_hbm.at[s, ci], idxpb.at[b],
                                     lsems.at[b])

    def unpack_src(b):
        row = idxpb.at[b]
        sb = sbufs.at[b]
        for j in range(CHUNK // 16):
            v = row[pl.ds(j * 16, 16)]
            sb[pl.ds(j * 16, 16)] = jnp.bitwise_and(v, 0xFFFF)

    def unpack_dst(b):
        row = idxpb.at[b]
        db = dbufs.at[b]
        for j in range(CHUNK // 16):
            v = row[pl.ds(j * 16, 16)]
            db[pl.ds(j * 16, 16)] = jnp.right_shift(v, 16)

    def gather(b):
        return pltpu.make_async_copy(h_sp.at[sbufs.at[b]], msgs.at[b],
                                     gsems.at[b])

    def scatter(b):
        return pltpu.make_async_copy(msgs.at[b], acc.at[dbufs.at[b]],
                                     ssems.at[b])

    load(0, 0).start()
    load(1, 1).start()
    load(0, 0).wait()
    unpack_src(0)
    unpack_dst(0)
    load(2, 0).start()
    load(1, 1).wait()
    unpack_src(1)
    gather(0).start()

    def body(k, carry):
        c0 = 2 * k
        c1 = c0 + 1
        more = k < GROUPS - 1

        gather(0).wait()
        scatter(0).start(add=True)

        @pl.when(more)
        def _():
            load(c0 + 2, 0).wait()
            unpack_src(0)

        @pl.when(k > 0)
        def _():
            scatter(1).wait()

        unpack_dst(1)

        @pl.when(more)
        def _():
            load(c1 + 2, 1).start()

        gather(1).start()
        gather(1).wait()
        scatter(1).start(add=True)

        @pl.when(more)
        def _():
            load(c1 + 2, 1).wait()
            unpack_src(1)

        scatter(0).wait()

        @pl.when(more)
        def _():
            unpack_dst(0)
            gather(0).start()

        @pl.when(k < GROUPS - 2)
        def _():
            load(c0 + 4, 0).start()

        return carry

    lax.fori_loop(0, GROUPS, body, 0)
    scatter(1).wait()
    plsc.subcore_barrier()

    pltpu.sync_copy(acc.at[pl.ds(r0, RPT)], out_hbm.at[c, pl.ds(r0, RPT)])

    @pl.when(s == 0)
    def _():
        pltpu.sync_copy(acc.at[pl.ds(REM0, N_NODES - REM0)],
                        out_hbm.at[c, pl.ds(REM0, N_NODES - REM0)])


@jax.jit
def kernel(x, edge_index, norm, weight, bias):
    x = x.astype(jnp.float32)
    norm = norm.astype(jnp.float32)

    src = edge_index[0].astype(jnp.int32)
    dst = edge_index[1].astype(jnp.int32)
    ppt = EDGES_PER_TILE - N_EDGES // NS
    src = jnp.concatenate(
        [src.reshape(NS, N_EDGES // NS),
         jnp.zeros((NS, ppt), jnp.int32)], axis=1)
    dpad = N_NODES + (jnp.arange(ppt, dtype=jnp.int32) % N_DUMMY)
    dst = jnp.concatenate(
        [dst.reshape(NS, N_EDGES // NS),
         jnp.broadcast_to(dpad, (NS, ppt))], axis=1)
    idxp = (src + dst * 65536).reshape(NS, CHUNKS_PER_TILE, CHUNK)

    w2 = jnp.moveaxis(weight.reshape(IN_CH, NC, HCH), 1, 0)
    h2 = _mm_call(x, norm, w2)

    zeros = jnp.zeros((N_ROWS, HCH), jnp.float32)
    mesh = plsc.VectorSubcoreMesh(core_axis_name="c", subcore_axis_name="s")
    acc2 = pl.kernel(
        _scatter_body,
        out_type=jax.ShapeDtypeStruct((NC, N_NODES, HCH), jnp.float32),
        mesh=mesh,
        compiler_params=pltpu.CompilerParams(use_tc_tiling_on_sc=False),
        scratch_types=[
            pltpu.VMEM_SHARED((N_ROWS, HCH), jnp.float32),
            pltpu.VMEM_SHARED((N_ROWS, HCH), jnp.float32),
            pltpu.VMEM((2, CHUNK), jnp.int32),
            pltpu.VMEM((2, CHUNK), jnp.int32),
            pltpu.VMEM((2, CHUNK), jnp.int32),
            pltpu.VMEM((2, CHUNK, HCH), jnp.float32),
            pltpu.SemaphoreType.DMA((2,)),
            pltpu.SemaphoreType.DMA((2,)),
            pltpu.SemaphoreType.DMA((2,)),
        ],
    )(h2, idxp, zeros)

    R = 2000
    out = pl.pallas_call(
        _finish_body,
        grid=(N_NODES // R,),
        in_specs=[
            pl.BlockSpec((NC, R, HCH), lambda i: (0, i, 0)),
            pl.BlockSpec((R, 1), lambda i: (i, 0)),
            pl.BlockSpec((1, OUT_CH), lambda i: (0, 0)),
        ],
        out_specs=pl.BlockSpec((R, OUT_CH), lambda i: (i, 0)),
        out_shape=jax.ShapeDtypeStruct((N_NODES, OUT_CH), jnp.float32),
    )(acc2, norm, bias.reshape(1, OUT_CH))
    return out


# speedup vs baseline: 1.2551x; 1.0250x over previous
import jax
import jax.numpy as jnp
from jax import lax
from jax.experimental import pallas as pl
from jax.experimental.pallas import tpu as pltpu
from jax.experimental.pallas import tpu_sc as plsc

N_NODES = 10000
N_EDGES = 320000
IN_CH = 128
OUT_CH = 128
HCH = OUT_CH // 2

NC = 2
NS = 16
CHUNK = 128

CHUNKS_PER_TILE = 158
EDGES_PER_TILE = CHUNKS_PER_TILE * CHUNK
E_PAD = EDGES_PER_TILE * NS
GROUPS = CHUNKS_PER_TILE // 2

N_DUMMY = 8
N_ROWS = N_NODES + N_DUMMY

RPT = 624
REM0 = RPT * NS


def _mm_body(x_ref, nrm_ref, w_ref, o_ref):
    o_ref[0] = jnp.dot(
        x_ref[...] * nrm_ref[...], w_ref[0],
        preferred_element_type=jnp.float32,
    )


def _finish_body(acc_ref, nrm_ref, b_ref, o_ref):
    o_ref[...] = (
        jnp.concatenate([acc_ref[0], acc_ref[1]], axis=1) * nrm_ref[...]
        + b_ref[...]
    )


def _mm_call(x, norm, w2):
    R = 5000
    return pl.pallas_call(
        _mm_body,
        grid=(NC, N_NODES // R),
        in_specs=[
            pl.BlockSpec((R, IN_CH), lambda j, i: (i, 0)),
            pl.BlockSpec((R, 1), lambda j, i: (i, 0)),
            pl.BlockSpec((1, IN_CH, HCH), lambda j, i: (j, 0, 0)),
        ],
        out_specs=pl.BlockSpec((1, R, HCH), lambda j, i: (j, i, 0)),
        out_shape=jax.ShapeDtypeStruct((NC, N_ROWS, HCH), jnp.float32),
    )(x, norm, w2)


def _scatter_body(h2_hbm, idxp_hbm, zeros_hbm, out_hbm,
                  h_sp, acc, idxpb, sbufs, dbufs, msgs, lsems, gsems, ssems):
    c = lax.axis_index("c")
    s = lax.axis_index("s")

    r0 = s * RPT
    pltpu.sync_copy(zeros_hbm.at[pl.ds(r0, RPT)], acc.at[pl.ds(r0, RPT)])
    pltpu.sync_copy(h2_hbm.at[c, pl.ds(r0, RPT)], h_sp.at[pl.ds(r0, RPT)])

    @pl.when(s == 0)
    def _():
        pltpu.sync_copy(zeros_hbm.at[pl.ds(REM0, N_ROWS - REM0)],
                        acc.at[pl.ds(REM0, N_ROWS - REM0)])
        pltpu.sync_copy(h2_hbm.at[c, pl.ds(REM0, N_ROWS - REM0)],
                        h_sp.at[pl.ds(REM0, N_ROWS - REM0)])

    plsc.subcore_barrier()

    def load(ci, b):
        return pltpu.make_async_copy(idxp_hbm.at[s, ci], idxpb.at[b],
                                     lsems.at[b])

    def unpack_src(b):
        row = idxpb.at[b]
        sb = sbufs.at[b]
        for j in range(CHUNK // 16):
            v = row[pl.ds(j * 16, 16)]
            sb[pl.ds(j * 16, 16)] = jnp.bitwise_and(v, 0xFFFF)

    def unpack_dst(b):
        row = idxpb.at[b]
        db = dbufs.at[b]
        for j in range(CHUNK // 16):
            v = row[pl.ds(j * 16, 16)]
            db[pl.ds(j * 16, 16)] = jnp.right_shift(v, 16)

    def gather(b):
        return pltpu.make_async_copy(h_sp.at[sbufs.at[b]], msgs.at[b],
                                     gsems.at[b])

    def scatter(b):
        return pltpu.make_async_copy(msgs.at[b], acc.at[dbufs.at[b]],
                                     ssems.at[b])

    load(0, 0).start()
    load(1, 1).start()
    load(0, 0).wait()
    unpack_src(0)
    unpack_dst(0)
    load(2, 0).start()
    load(1, 1).wait()
    unpack_src(1)
    gather(0).start()

    def body(k, carry):
        c0 = 2 * k
        c1 = c0 + 1
        more = k < GROUPS - 1

        gather(0).wait()
        scatter(0).start(add=True)

        @pl.when(more)
        def _():
            load(c0 + 2, 0).wait()
            unpack_src(0)

        @pl.when(k > 0)
        def _():
            scatter(1).wait()

        unpack_dst(1)

        @pl.when(more)
        def _():
            load(c1 + 2, 1).start()

        gather(1).start()
        gather(1).wait()
        scatter(1).start(add=True)

        @pl.when(more)
        def _():
            load(c1 + 2, 1).wait()
            unpack_src(1)

        scatter(0).wait()

        @pl.when(more)
        def _():
            unpack_dst(0)
            gather(0).start()

        @pl.when(k < GROUPS - 2)
        def _():
            load(c0 + 4, 0).start()

        return carry

    lax.fori_loop(0, GROUPS, body, 0)
    scatter(1).wait()
    plsc.subcore_barrier()

    pltpu.sync_copy(acc.at[pl.ds(r0, RPT)], out_hbm.at[c, pl.ds(r0, RPT)])

    @pl.when(s == 0)
    def _():
        pltpu.sync_copy(acc.at[pl.ds(REM0, N_NODES - REM0)],
                        out_hbm.at[c, pl.ds(REM0, N_NODES - REM0)])


@jax.jit
def kernel(x, edge_index, norm, weight, bias):
    x = x.astype(jnp.float32)
    norm = norm.astype(jnp.float32)

    src = edge_index[0].astype(jnp.int32)
    dst = edge_index[1].astype(jnp.int32)
    ppt = EDGES_PER_TILE - N_EDGES // NS
    src = jnp.concatenate(
        [src.reshape(NS, N_EDGES // NS),
         jnp.zeros((NS, ppt), jnp.int32)], axis=1)
    dpad = N_NODES + (jnp.arange(ppt, dtype=jnp.int32) % N_DUMMY)
    dst = jnp.concatenate(
        [dst.reshape(NS, N_EDGES // NS),
         jnp.broadcast_to(dpad, (NS, ppt))], axis=1)
    idxp = (src + dst * 65536).reshape(NS, CHUNKS_PER_TILE, CHUNK)

    w2 = jnp.moveaxis(weight.reshape(IN_CH, NC, HCH), 1, 0)
    h2 = _mm_call(x, norm, w2)

    zeros = jnp.zeros((N_ROWS, HCH), jnp.float32)
    mesh = plsc.VectorSubcoreMesh(core_axis_name="c", subcore_axis_name="s")
    acc2 = pl.kernel(
        _scatter_body,
        out_type=jax.ShapeDtypeStruct((NC, N_NODES, HCH), jnp.float32),
        mesh=mesh,
        compiler_params=pltpu.CompilerParams(use_tc_tiling_on_sc=False),
        scratch_types=[
            pltpu.VMEM_SHARED((N_ROWS, HCH), jnp.float32),
            pltpu.VMEM_SHARED((N_ROWS, HCH), jnp.float32),
            pltpu.VMEM((2, CHUNK), jnp.int32),
            pltpu.VMEM((2, CHUNK), jnp.int32),
            pltpu.VMEM((2, CHUNK), jnp.int32),
            pltpu.VMEM((2, CHUNK, HCH), jnp.float32),
            pltpu.SemaphoreType.DMA((2,)),
            pltpu.SemaphoreType.DMA((2,)),
            pltpu.SemaphoreType.DMA((2,)),
        ],
    )(h2, idxp, zeros)

    R = 5000
    out = pl.pallas_call(
        _finish_body,
        grid=(N_NODES // R,),
        in_specs=[
            pl.BlockSpec((NC, R, HCH), lambda i: (0, i, 0)),
            pl.BlockSpec((R, 1), lambda i: (i, 0)),
            pl.BlockSpec((1, OUT_CH), lambda i: (0, 0)),
        ],
        out_specs=pl.BlockSpec((R, OUT_CH), lambda i: (i, 0)),
        out_shape=jax.ShapeDtypeStruct((N_NODES, OUT_CH), jnp.float32),
    )(acc2, norm, bias.reshape(1, OUT_CH))
    return out
